# Initial kernel scaffold; baseline (speedup 1.0000x reference)
import functools

import jax
import jax.numpy as jnp
from jax import lax
from jax.experimental import pallas as pl
from jax.experimental.pallas import tpu as pltpu
from jax.experimental.pallas import tpu_sc as plsc

N = 50000
E = 1600000
D_EDGE = 4
HID = 16
C_OUT = 24
PAD = 32

NC = 2
NS = 16
NW = NC * NS
EPW = E // NW
CHUNK = 2000
NCHUNK = EPW // CHUNK
GROUPS = CHUNK // 16

RPT = N // NS
ZROWS = 625
ZCOPIES = RPT // ZROWS

_mesh = plsc.VectorSubcoreMesh(core_axis_name="c", subcore_axis_name="s")


@functools.partial(
    pl.kernel,
    out_type=jax.ShapeDtypeStruct((E,), jnp.float32),
    mesh=_mesh,
    scratch_types=[
        pltpu.VMEM((N,), jnp.float32),
        pltpu.VMEM((CHUNK,), jnp.int32),
        pltpu.VMEM((CHUNK,), jnp.float32),
    ],
)
def _sc_gather(x_hbm, src_hbm, out_hbm, x_v, idx_v, xs_v):
    wid = lax.axis_index("s") * NC + lax.axis_index("c")
    base = wid * EPW
    pltpu.sync_copy(x_hbm, x_v)

    def chunk_body(ci, carry):
        off = base + ci * CHUNK
        pltpu.sync_copy(src_hbm.at[pl.ds(off, CHUNK)], idx_v)

        def grp(gi, c):
            idx = idx_v[pl.ds(gi * 16, 16)]
            xs_v[pl.ds(gi * 16, 16)] = plsc.load_gather(x_v, [idx])
            return c

        lax.fori_loop(0, GROUPS, grp, 0)
        pltpu.sync_copy(xs_v, out_hbm.at[pl.ds(off, CHUNK)])
        return carry

    lax.fori_loop(0, NCHUNK, chunk_body, 0)


@functools.partial(
    pl.kernel,
    out_type=jax.ShapeDtypeStruct((NC, N, PAD), jnp.float32),
    mesh=_mesh,
    scratch_types=[
        pltpu.VMEM((CHUNK, PAD), jnp.float32),
        pltpu.VMEM((CHUNK,), jnp.int32),
        pltpu.VMEM((ZROWS, PAD), jnp.float32),
        pltpu.VMEM_SHARED((N, PAD), jnp.float32),
    ],
)
def _sc_scatter(msg_hbm, dst_hbm, out_hbm, msg_v, dst_v, z_v, acc_sh):
    cid = lax.axis_index("c")
    sid = lax.axis_index("s")
    wid = sid * NC + cid
    base = wid * EPW

    zeros16 = jnp.zeros((16,), jnp.float32)

    def zrow(r, c):
        z_v[r, pl.ds(0, 16)] = zeros16
        z_v[r, pl.ds(16, 16)] = zeros16
        return c

    lax.fori_loop(0, ZROWS, zrow, 0)

    def zcopy(j, c):
        pltpu.sync_copy(z_v, acc_sh.at[pl.ds(sid * RPT + j * ZROWS, ZROWS)])
        return c

    lax.fori_loop(0, ZCOPIES, zcopy, 0)
    plsc.subcore_barrier()

    def chunk_body(ci, carry):
        off = base + ci * CHUNK
        pltpu.sync_copy(dst_hbm.at[pl.ds(off, CHUNK)], dst_v)
        pltpu.sync_copy(msg_hbm.at[pl.ds(off, CHUNK)], msg_v)
        pltpu.sync_copy(msg_v, acc_sh.at[dst_v], add=True)
        return carry

    lax.fori_loop(0, NCHUNK, chunk_body, 0)
    plsc.subcore_barrier()
    pltpu.sync_copy(
        acc_sh.at[pl.ds(sid * RPT, RPT)],
        out_hbm.at[cid, pl.ds(sid * RPT, RPT)],
    )


BE = 12800


def _tc_msg_body(ea_ref, xs_ref, w0_ref, b0_ref, w1_ref, b1_ref, out_ref):
    h = jnp.maximum(
        jnp.dot(ea_ref[...], w0_ref[...], preferred_element_type=jnp.float32)
        + b0_ref[...],
        0.0,
    )
    theta = (
        jnp.dot(h, w1_ref[...], preferred_element_type=jnp.float32)
        + b1_ref[...]
    )
    msg = theta * xs_ref[...]
    out_ref[...] = jnp.concatenate(
        [
            msg,
            jnp.ones((BE, 1), jnp.float32),
            jnp.zeros((BE, PAD - C_OUT - 1), jnp.float32),
        ],
        axis=1,
    )


_tc_msg = pl.pallas_call(
    _tc_msg_body,
    grid=(E // BE,),
    in_specs=[
        pl.BlockSpec((BE, D_EDGE), lambda i: (i, 0)),
        pl.BlockSpec((BE, 1), lambda i: (i, 0)),
        pl.BlockSpec((D_EDGE, HID), lambda i: (0, 0)),
        pl.BlockSpec((1, HID), lambda i: (0, 0)),
        pl.BlockSpec((HID, C_OUT), lambda i: (0, 0)),
        pl.BlockSpec((1, C_OUT), lambda i: (0, 0)),
    ],
    out_specs=pl.BlockSpec((BE, PAD), lambda i: (i, 0)),
    out_shape=jax.ShapeDtypeStruct((E, PAD), jnp.float32),
)


BN = 2000


def _tc_final_body(p0_ref, p1_ref, x_ref, root_ref, bias_ref, out_ref):
    p0 = p0_ref[...]
    p1 = p1_ref[...]
    s = p0[:, :C_OUT] + p1[:, :C_OUT]
    cnt = p0[:, C_OUT:C_OUT + 1] + p1[:, C_OUT:C_OUT + 1]
    mean = s / jnp.maximum(cnt, 1.0)
    out_ref[...] = mean + x_ref[...] * root_ref[...] + bias_ref[...]


_tc_final = pl.pallas_call(
    _tc_final_body,
    grid=(N // BN,),
    in_specs=[
        pl.BlockSpec((BN, PAD), lambda i: (i, 0)),
        pl.BlockSpec((BN, PAD), lambda i: (i, 0)),
        pl.BlockSpec((BN, 1), lambda i: (i, 0)),
        pl.BlockSpec((1, C_OUT), lambda i: (0, 0)),
        pl.BlockSpec((1, C_OUT), lambda i: (0, 0)),
    ],
    out_specs=pl.BlockSpec((BN, C_OUT), lambda i: (i, 0)),
    out_shape=jax.ShapeDtypeStruct((N, C_OUT), jnp.float32),
)


def kernel(x, edge_index, edge_attr, w0, b0, w1, b1, root, bias):
    src = edge_index[0]
    dst = edge_index[1]
    xsrc = _sc_gather(x.reshape(N), src)
    msg = _tc_msg(
        edge_attr,
        xsrc.reshape(E, 1),
        w0,
        b0.reshape(1, HID),
        w1,
        b1.reshape(1, C_OUT),
    )
    partials = _sc_scatter(msg, dst)
    out = _tc_final(
        partials[0],
        partials[1],
        x,
        root,
        bias.reshape(1, C_OUT),
    )
    return out


# speedup vs baseline: 5.7042x; 5.7042x over previous
import functools

import jax
import jax.numpy as jnp
from jax import lax
from jax.experimental import pallas as pl
from jax.experimental.pallas import tpu as pltpu
from jax.experimental.pallas import tpu_sc as plsc

N = 50000
E = 1600000
D_EDGE = 4
HID = 16
C_OUT = 24
W = 16

NC = 2
NS = 16
NW = NC * NS
EPW = E // NW
CHUNK = 2000
NCHUNK = EPW // CHUNK
GROUPS = CHUNK // 16

NPAD = 50176
RPT = NPAD // NS
ZROWS = 784
ZCOPIES = RPT // ZROWS

_mesh = plsc.VectorSubcoreMesh(core_axis_name="c", subcore_axis_name="s")
_sc_params = pltpu.CompilerParams(
    needs_layout_passes=False, use_tc_tiling_on_sc=False
)


@functools.partial(
    pl.kernel,
    out_type=jax.ShapeDtypeStruct((E,), jnp.float32),
    mesh=_mesh,
    compiler_params=_sc_params,
    scratch_types=[
        pltpu.VMEM((N,), jnp.float32),
        pltpu.VMEM((CHUNK,), jnp.int32),
        pltpu.VMEM((CHUNK,), jnp.float32),
    ],
)
def _sc_gather(x_hbm, src_hbm, out_hbm, x_v, idx_v, xs_v):
    wid = lax.axis_index("s") * NC + lax.axis_index("c")
    base = wid * EPW
    pltpu.sync_copy(x_hbm, x_v)

    def chunk_body(ci, carry):
        off = base + ci * CHUNK
        pltpu.sync_copy(src_hbm.at[pl.ds(off, CHUNK)], idx_v)

        def grp(gi, c):
            idx = idx_v[pl.ds(gi * 16, 16)]
            xs_v[pl.ds(gi * 16, 16)] = plsc.load_gather(x_v, [idx])
            return c

        lax.fori_loop(0, GROUPS, grp, 0)
        pltpu.sync_copy(xs_v, out_hbm.at[pl.ds(off, CHUNK)])
        return carry

    lax.fori_loop(0, NCHUNK, chunk_body, 0)


@functools.partial(
    pl.kernel,
    out_type=(
        jax.ShapeDtypeStruct((NC, NPAD, W), jnp.float32),
        jax.ShapeDtypeStruct((NC, NPAD, W), jnp.float32),
    ),
    mesh=_mesh,
    compiler_params=_sc_params,
    scratch_types=[
        pltpu.VMEM((CHUNK, W), jnp.float32),
        pltpu.VMEM((CHUNK,), jnp.int32),
        pltpu.VMEM((ZROWS, W), jnp.float32),
        pltpu.VMEM_SHARED((NPAD, W), jnp.float32),
    ],
)
def _sc_scatter(lo_hbm, hi_hbm, dst_hbm, outlo_hbm, outhi_hbm,
                msg_v, dst_v, z_v, acc_sh):
    cid = lax.axis_index("c")
    sid = lax.axis_index("s")
    wid = sid * NC + cid
    base = wid * EPW

    zeros16 = jnp.zeros((16,), jnp.float32)

    def zrow(r, c):
        z_v[r, pl.ds(0, 16)] = zeros16
        return c

    lax.fori_loop(0, ZROWS, zrow, 0)

    for msg_hbm, out_hbm in ((lo_hbm, outlo_hbm), (hi_hbm, outhi_hbm)):
        def zcopy(j, c):
            pltpu.sync_copy(
                z_v, acc_sh.at[pl.ds(sid * RPT + j * ZROWS, ZROWS)]
            )
            return c

        lax.fori_loop(0, ZCOPIES, zcopy, 0)
        plsc.subcore_barrier()

        def chunk_body(ci, carry):
            off = base + ci * CHUNK
            pltpu.sync_copy(dst_hbm.at[pl.ds(off, CHUNK)], dst_v)
            pltpu.sync_copy(msg_hbm.at[pl.ds(off, CHUNK)], msg_v)
            pltpu.sync_copy(msg_v, acc_sh.at[dst_v], add=True)
            return carry

        lax.fori_loop(0, NCHUNK, chunk_body, 0)
        plsc.subcore_barrier()
        pltpu.sync_copy(
            acc_sh.at[pl.ds(sid * RPT, RPT)],
            out_hbm.at[cid, pl.ds(sid * RPT, RPT)],
        )


BE = 12800


def _tc_msg_body(ea_ref, xs_ref, w0_ref, b0_ref, w1_ref, b1_ref,
                 lo_ref, hi_ref):
    h = jnp.maximum(
        jnp.dot(ea_ref[...], w0_ref[...], preferred_element_type=jnp.float32)
        + b0_ref[...],
        0.0,
    )
    theta = (
        jnp.dot(h, w1_ref[...], preferred_element_type=jnp.float32)
        + b1_ref[...]
    )
    msg = theta * xs_ref[...]
    lo_ref[...] = msg[:, :W]
    hi_ref[...] = jnp.concatenate(
        [
            msg[:, W:C_OUT],
            jnp.ones((BE, 1), jnp.float32),
            jnp.zeros((BE, 2 * W - C_OUT - 1), jnp.float32),
        ],
        axis=1,
    )


_tc_msg = pl.pallas_call(
    _tc_msg_body,
    grid=(E // BE,),
    in_specs=[
        pl.BlockSpec((BE, D_EDGE), lambda i: (i, 0)),
        pl.BlockSpec((BE, 1), lambda i: (i, 0)),
        pl.BlockSpec((D_EDGE, HID), lambda i: (0, 0)),
        pl.BlockSpec((1, HID), lambda i: (0, 0)),
        pl.BlockSpec((HID, C_OUT), lambda i: (0, 0)),
        pl.BlockSpec((1, C_OUT), lambda i: (0, 0)),
    ],
    out_specs=(
        pl.BlockSpec((BE, W), lambda i: (i, 0)),
        pl.BlockSpec((BE, W), lambda i: (i, 0)),
    ),
    out_shape=(
        jax.ShapeDtypeStruct((E, W), jnp.float32),
        jax.ShapeDtypeStruct((E, W), jnp.float32),
    ),
)


BN = 2000


def _tc_final_body(plo_ref, phi_ref, x_ref, root_ref, bias_ref, out_ref):
    lo = plo_ref[0] + plo_ref[1]
    hi = phi_ref[0] + phi_ref[1]
    s = jnp.concatenate([lo, hi[:, : C_OUT - W]], axis=1)
    cnt = hi[:, C_OUT - W:C_OUT - W + 1]
    mean = s / jnp.maximum(cnt, 1.0)
    out_ref[...] = mean + x_ref[...] * root_ref[...] + bias_ref[...]


_tc_final = pl.pallas_call(
    _tc_final_body,
    grid=(N // BN,),
    in_specs=[
        pl.BlockSpec((NC, BN, W), lambda i: (0, i, 0)),
        pl.BlockSpec((NC, BN, W), lambda i: (0, i, 0)),
        pl.BlockSpec((BN, 1), lambda i: (i, 0)),
        pl.BlockSpec((1, C_OUT), lambda i: (0, 0)),
        pl.BlockSpec((1, C_OUT), lambda i: (0, 0)),
    ],
    out_specs=pl.BlockSpec((BN, C_OUT), lambda i: (i, 0)),
    out_shape=jax.ShapeDtypeStruct((N, C_OUT), jnp.float32),
)


def kernel(x, edge_index, edge_attr, w0, b0, w1, b1, root, bias):
    src = edge_index[0]
    dst = edge_index[1]
    xsrc = _sc_gather(x.reshape(N), src)
    msg_lo, msg_hi = _tc_msg(
        edge_attr,
        xsrc.reshape(E, 1),
        w0,
        b0.reshape(1, HID),
        w1,
        b1.reshape(1, C_OUT),
    )
    p_lo, p_hi = _sc_scatter(msg_lo, msg_hi, dst)
    out = _tc_final(
        p_lo,
        p_hi,
        x,
        root,
        bias.reshape(1, C_OUT),
    )
    return out


# speedup vs baseline: 6.2768x; 1.1004x over previous
import functools

import jax
import jax.numpy as jnp
from jax import lax
from jax.experimental import pallas as pl
from jax.experimental.pallas import tpu as pltpu
from jax.experimental.pallas import tpu_sc as plsc

N = 50000
E = 1600000
D_EDGE = 4
HID = 16
C_OUT = 24
W = 16
CNT_COL = 8

NC = 2
NS = 16
NW = NC * NS
EPW = E // NW
CHUNK = 400
NCHUNK = EPW // CHUNK
GROUPS = CHUNK // 16

NPAD = 50176
RPT = NPAD // NS
ZROWS = 196
ZCOPIES = RPT // ZROWS

_mesh = plsc.VectorSubcoreMesh(core_axis_name="c", subcore_axis_name="s")
_sc_params = pltpu.CompilerParams(
    needs_layout_passes=False, use_tc_tiling_on_sc=False
)

@functools.partial(
    pl.kernel,
    out_type=(
        jax.ShapeDtypeStruct((NC, NPAD, W), jnp.float32),
        jax.ShapeDtypeStruct((NC, NPAD, W), jnp.float32),
    ),
    mesh=_mesh,
    compiler_params=_sc_params,
    scratch_types=[
        pltpu.VMEM((N,), jnp.float32),
        pltpu.VMEM((CHUNK, W), jnp.float32),
        pltpu.VMEM((CHUNK,), jnp.int32),
        pltpu.VMEM((CHUNK,), jnp.int32),
        pltpu.VMEM((ZROWS, W), jnp.float32),
        pltpu.VMEM_SHARED((NPAD, W), jnp.float32),
    ],
)
def _sc_scatter(x_hbm, lo_hbm, hi_hbm, src_hbm, dst_hbm,
                outlo_hbm, outhi_hbm,
                x_v, msg_v, src_v, dst_v, z_v, acc_sh):
    cid = lax.axis_index("c")
    sid = lax.axis_index("s")
    wid = sid * NC + cid
    base = wid * EPW

    pltpu.sync_copy(x_hbm, x_v)

    zeros16 = jnp.zeros((16,), jnp.float32)

    def zrow(r, c):
        z_v[r, pl.ds(0, 16)] = zeros16
        return c

    lax.fori_loop(0, ZROWS, zrow, 0)

    for phase, (msg_hbm, out_hbm) in enumerate(
        ((lo_hbm, outlo_hbm), (hi_hbm, outhi_hbm))
    ):
        def zcopy(j, c):
            pltpu.sync_copy(
                z_v, acc_sh.at[pl.ds(sid * RPT + j * ZROWS, ZROWS)]
            )
            return c

        lax.fori_loop(0, ZCOPIES, zcopy, 0)
        plsc.subcore_barrier()

        def chunk_body(ci, carry):
            off = base + ci * CHUNK
            pltpu.sync_copy(src_hbm.at[pl.ds(off, CHUNK)], src_v)
            pltpu.sync_copy(dst_hbm.at[pl.ds(off, CHUNK)], dst_v)
            pltpu.sync_copy(msg_hbm.at[pl.ds(off, CHUNK)], msg_v)

            def grp(gi, c):
                idx = src_v[pl.ds(gi * 16, 16)]
                xs = plsc.load_gather(x_v, [idx])
                for b in range(16):
                    sp = lax.gather(
                        xs,
                        jnp.full((16, 1), b, jnp.int32),
                        lax.GatherDimensionNumbers(
                            offset_dims=(),
                            collapsed_slice_dims=(0,),
                            start_index_map=(0,),
                        ),
                        (1,),
                        mode=lax.GatherScatterMode.PROMISE_IN_BOUNDS,
                    )
                    if phase == 1:
                        lane = lax.iota(jnp.int32, 16)
                        sp = jnp.where(lane == CNT_COL, 1.0, sp)
                    row = gi * 16 + b
                    msg_v[row, pl.ds(0, 16)] = (
                        msg_v[row, pl.ds(0, 16)] * sp
                    )
                return c

            lax.fori_loop(0, GROUPS, grp, 0)
            pltpu.sync_copy(msg_v, acc_sh.at[dst_v], add=True)
            return carry

        lax.fori_loop(0, NCHUNK, chunk_body, 0)
        plsc.subcore_barrier()
        pltpu.sync_copy(
            acc_sh.at[pl.ds(sid * RPT, RPT)],
            out_hbm.at[cid, pl.ds(sid * RPT, RPT)],
        )


BE = 12800


def _tc_msg_body(ea_ref, w0_ref, b0_ref, w1_ref, b1_ref, lo_ref, hi_ref):
    h = jnp.maximum(
        jnp.dot(ea_ref[...], w0_ref[...], preferred_element_type=jnp.float32)
        + b0_ref[...],
        0.0,
    )
    theta = (
        jnp.dot(h, w1_ref[...], preferred_element_type=jnp.float32)
        + b1_ref[...]
    )
    lo_ref[...] = theta[:, :W]
    hi_ref[...] = jnp.concatenate(
        [
            theta[:, W:C_OUT],
            jnp.ones((BE, 1), jnp.float32),
            jnp.zeros((BE, 2 * W - C_OUT - 1), jnp.float32),
        ],
        axis=1,
    )


_tc_msg = pl.pallas_call(
    _tc_msg_body,
    grid=(E // BE,),
    in_specs=[
        pl.BlockSpec((BE, D_EDGE), lambda i: (i, 0)),
        pl.BlockSpec((D_EDGE, HID), lambda i: (0, 0)),
        pl.BlockSpec((1, HID), lambda i: (0, 0)),
        pl.BlockSpec((HID, C_OUT), lambda i: (0, 0)),
        pl.BlockSpec((1, C_OUT), lambda i: (0, 0)),
    ],
    out_specs=(
        pl.BlockSpec((BE, W), lambda i: (i, 0)),
        pl.BlockSpec((BE, W), lambda i: (i, 0)),
    ),
    out_shape=(
        jax.ShapeDtypeStruct((E, W), jnp.float32),
        jax.ShapeDtypeStruct((E, W), jnp.float32),
    ),
)


BN = 2000


def _tc_final_body(plo_ref, phi_ref, x_ref, root_ref, bias_ref, out_ref):
    lo = plo_ref[0] + plo_ref[1]
    hi = phi_ref[0] + phi_ref[1]
    s = jnp.concatenate([lo, hi[:, : C_OUT - W]], axis=1)
    cnt = hi[:, CNT_COL:CNT_COL + 1]
    mean = s / jnp.maximum(cnt, 1.0)
    out_ref[...] = mean + x_ref[...] * root_ref[...] + bias_ref[...]


_tc_final = pl.pallas_call(
    _tc_final_body,
    grid=(N // BN,),
    in_specs=[
        pl.BlockSpec((NC, BN, W), lambda i: (0, i, 0)),
        pl.BlockSpec((NC, BN, W), lambda i: (0, i, 0)),
        pl.BlockSpec((BN, 1), lambda i: (i, 0)),
        pl.BlockSpec((1, C_OUT), lambda i: (0, 0)),
        pl.BlockSpec((1, C_OUT), lambda i: (0, 0)),
    ],
    out_specs=pl.BlockSpec((BN, C_OUT), lambda i: (i, 0)),
    out_shape=jax.ShapeDtypeStruct((N, C_OUT), jnp.float32),
)


def kernel(x, edge_index, edge_attr, w0, b0, w1, b1, root, bias):
    src = edge_index[0]
    dst = edge_index[1]
    theta_lo, theta_hi = _tc_msg(
        edge_attr,
        w0,
        b0.reshape(1, HID),
        w1,
        b1.reshape(1, C_OUT),
    )
    p_lo, p_hi = _sc_scatter(x.reshape(N), theta_lo, theta_hi, src, dst)
    out = _tc_final(
        p_lo,
        p_hi,
        x,
        root,
        bias.reshape(1, C_OUT),
    )
    return out
